# baseline (speedup 1.0000x reference)
import functools

import jax
import jax.numpy as jnp
from jax import lax
from jax.experimental import pallas as pl
from jax.experimental.pallas import tpu as pltpu
from jax.experimental.pallas import tpu_sc as plsc

_B, _N, _G, _M = 4, 8192, 256, 32
_D_ENC, _D_TR, _D_EMB = 512, 768, 512
_DEPTH, _HEADS = 12, 12
_HD = _D_TR // _HEADS
_T = _G + 1
_TP = 264
_NR, _NL = 64, 128
_NEG = -1e30
_INF = 1e30


def _fps_body(px_ref, py_ref, pz_ref, cen_ref):
    px = px_ref[...]
    py = py_ref[...]
    pz = pz_ref[...]
    rows8 = lax.broadcasted_iota(jnp.int32, (_B, _G, 8), 1)
    cols8 = lax.broadcasted_iota(jnp.int32, (_B, _G, 8), 2)

    def red(a, op):
        return op(op(a, axis=2, keepdims=True), axis=1, keepdims=True)

    def cen_row(cx, cy, cz):
        return jnp.where(cols8 == 0, cx,
               jnp.where(cols8 == 1, cy,
               jnp.where(cols8 == 2, cz, 0.0)))

    c0x = px[:, 0:1, 0:1]
    c0y = py[:, 0:1, 0:1]
    c0z = pz[:, 0:1, 0:1]
    dists0 = (px - c0x) ** 2 + (py - c0y) ** 2 + (pz - c0z) ** 2
    cen0 = jnp.where(rows8 == 0, cen_row(c0x, c0y, c0z),
                     jnp.zeros((_B, _G, 8), jnp.float32))

    def fps_step(i, carry):
        dists, cen = carry
        m = red(dists, jnp.max)
        onehot = dists == m
        cx = red(jnp.where(onehot, px, 0.0), jnp.sum)
        cy = red(jnp.where(onehot, py, 0.0), jnp.sum)
        cz = red(jnp.where(onehot, pz, 0.0), jnp.sum)
        cen = jnp.where(rows8 == i, cen_row(cx, cy, cz), cen)
        dnew = (px - cx) ** 2 + (py - cy) ** 2 + (pz - cz) ** 2
        return jnp.minimum(dists, dnew), cen

    _, cen = lax.fori_loop(1, _G, fps_step, (dists0, cen0))
    cen_ref[...] = cen


def _fps(px, py, pz):
    full = lambda s: pl.BlockSpec(s, lambda: tuple(0 for _ in s))
    return pl.pallas_call(
        _fps_body,
        in_specs=[full((_B, _NR, _NL))] * 3,
        out_specs=full((_B, _G, 8)),
        out_shape=jax.ShapeDtypeStruct((_B, _G, 8), jnp.float32),
    )(px, py, pz)


def _knn_body(ptsT_ref, cen_ref, idx_ref):
    b = pl.program_id(0)
    cenT = cen_ref[0]
    P8 = ptsT_ref[0]
    pn = jnp.sum(P8 * P8, axis=0, keepdims=True)
    cn = jnp.sum(cenT * cenT, axis=1, keepdims=True)
    cp = lax.dot_general(cenT, P8, (((1,), (0,)), ((), ())),
                         preferred_element_type=jnp.float32)
    D = pn + cn - 2.0 * cp

    coli = lax.broadcasted_iota(jnp.int32, (_G, _N), 1)
    lane = lax.broadcasted_iota(jnp.int32, (_G, _NL), 1)
    off = b * _N

    K = lax.bitcast_convert_type(D + 1.0, jnp.int32)
    maxi = jnp.int32(0x7FFFFFFF)

    def knn_step(k, carry):
        lastk, idxbuf = carry
        m = jnp.min(jnp.where(K > lastk, K, maxi),
                    axis=1, keepdims=True)
        rowidx = jnp.min(jnp.where(K == m, coli, _N),
                         axis=1, keepdims=True)
        idxbuf = jnp.where(lane == k, rowidx + off, idxbuf)
        return m, idxbuf

    _, idxbuf = lax.fori_loop(
        0, _M, knn_step,
        (jnp.full((_G, 1), jnp.iinfo(jnp.int32).min, jnp.int32),
         jnp.zeros((_G, _NL), jnp.int32)))
    idx_ref[0] = idxbuf


def _knn(ptsT, cen8):
    return pl.pallas_call(
        _knn_body,
        grid=(_B,),
        in_specs=[
            pl.BlockSpec((1, 8, _N), lambda b: (b, 0, 0)),
            pl.BlockSpec((1, _G, 8), lambda b: (b, 0, 0)),
        ],
        out_specs=pl.BlockSpec((1, _G, _NL), lambda b: (b, 0, 0)),
        out_shape=jax.ShapeDtypeStruct((_B, _G, _NL), jnp.int32),
    )(ptsT, cen8)


def _sc_gather(table, idx):
    tot = idx.shape[0]
    info = plsc.get_sparse_core_info()
    nc = info.num_cores
    nw = nc * info.num_subcores
    bpw = tot // nw
    chunks = bpw // 128
    idx3 = idx.reshape(nw, chunks, 128)
    mesh = plsc.VectorSubcoreMesh(core_axis_name="c", subcore_axis_name="s")

    @functools.partial(
        pl.kernel,
        mesh=mesh,
        compiler_params=pltpu.CompilerParams(use_tc_tiling_on_sc=False),
        out_type=jax.ShapeDtypeStruct((tot, 16), jnp.float32),
        scratch_types=[
            pltpu.VMEM((chunks, 128), jnp.int32),
            pltpu.VMEM((bpw, 16), jnp.float32),
            pltpu.SemaphoreType.DMA,
        ],
    )
    def k(table_hbm, idx_hbm, out_hbm, idx_v, rows_v, sem):
        wid = lax.axis_index("s") * nc + lax.axis_index("c")
        pltpu.sync_copy(idx_hbm.at[wid], idx_v)
        copies = []
        for j in range(chunks):
            copies.append(pltpu.async_copy(
                table_hbm.at[idx_v.at[j]],
                rows_v.at[pl.ds(j * 128, 128)], sem))
        for c in copies:
            c.wait()
        pltpu.sync_copy(rows_v, out_hbm.at[pl.ds(wid * bpw, bpw)])

    return k(table, idx3)


_GB = 128
_ROWS = _GB * _M


def _enc_body(feat_ref, cen_ref, w1_ref, b1_ref, g1_ref, bb1_ref,
              w2_ref, b2_ref, w3_ref, b3_ref, g2_ref, bb2_ref,
              w4_ref, b4_ref, e2tw_ref, e2tb_ref,
              pw1_ref, pb1_ref, pw2_ref, pb2_ref, out_ref):
    x = feat_ref[...]
    c = cen_ref[...]
    x = (x.reshape(_GB, _M, 16) - c[:, None, :]).reshape(_ROWS, 16)
    f = jnp.dot(x, w1_ref[...], preferred_element_type=jnp.float32) + b1_ref[...]
    f = jax.nn.relu(f * g1_ref[...] + bb1_ref[...])
    f = jnp.dot(f, w2_ref[...], preferred_element_type=jnp.float32) + b2_ref[...]
    fg = jnp.max(f.reshape(_GB, _M, 256), axis=1)
    fgb = jnp.broadcast_to(fg[:, None, :], (_GB, _M, 256)).reshape(_ROWS, 256)
    f = jnp.concatenate([fgb, f], axis=-1)
    f = jnp.dot(f, w3_ref[...], preferred_element_type=jnp.float32) + b3_ref[...]
    f = jax.nn.relu(f * g2_ref[...] + bb2_ref[...])
    f = jnp.dot(f, w4_ref[...], preferred_element_type=jnp.float32) + b4_ref[...]
    tok = jnp.max(f.reshape(_GB, _M, _D_ENC), axis=1)
    tok = jnp.dot(tok, e2tw_ref[...], preferred_element_type=jnp.float32) + e2tb_ref[...]
    ph = jax.nn.gelu(jnp.dot(c, pw1_ref[...], preferred_element_type=jnp.float32)
                     + pb1_ref[...])
    pos = jnp.dot(ph, pw2_ref[...], preferred_element_type=jnp.float32) + pb2_ref[...]
    out_ref[...] = tok + pos


def _encoder(feats, cen16, wts):
    nsteps = (_B * _G) // _GB
    const = lambda shape: pl.BlockSpec(shape, lambda i: tuple(0 for _ in shape))
    in_specs = [
        pl.BlockSpec((_ROWS, 16), lambda i: (i, 0)),
        pl.BlockSpec((_GB, 16), lambda i: (i, 0)),
    ] + [const(w.shape) for w in wts]
    return pl.pallas_call(
        _enc_body,
        grid=(nsteps,),
        in_specs=in_specs,
        out_specs=pl.BlockSpec((_GB, _D_TR), lambda i: (i, 0)),
        out_shape=jax.ShapeDtypeStruct((_B * _G, _D_TR), jnp.float32),
    )(feats, cen16, *wts)


def _ln(x, g, b):
    m = jnp.mean(x, axis=-1, keepdims=True)
    v = jnp.mean((x - m) ** 2, axis=-1, keepdims=True)
    return (x - m) / jnp.sqrt(v + 1e-5) * g + b


_KJ = 4
_HCH = 4 * _D_TR // _KJ


def _tr_body(x0_ref, l1g_ref, l1b_ref, qkvw_ref, qkvb_ref, pw_ref, pb_ref,
             l2g_ref, l2b_ref, m1w_ref, m1b_ref, m2w_ref, m2b_ref,
             ng_ref, nb_ref, t2ew_ref, t2eb_ref, out_ref, x_ref, acc_ref):
    d = pl.program_id(0)
    j = pl.program_id(1)

    @pl.when((d == 0) & (j == 0))
    def _():
        x_ref[...] = x0_ref[...]

    @pl.when(j == 0)
    def _():
        x = x_ref[...]
        keymask = jnp.where(
            lax.broadcasted_iota(jnp.int32, (1, _TP), 1) >= _T, _NEG, 0.0)
        h = _ln(x, l1g_ref[0], l1b_ref[0])
        y = jnp.dot(h, qkvw_ref[0], preferred_element_type=jnp.float32) \
            + qkvb_ref[0]
        brows = []
        for b in range(_B):
            r0 = b * _TP
            heads = []
            for hh in range(_HEADS):
                q = y[r0:r0 + _TP, hh * _HD:(hh + 1) * _HD]
                k = y[r0:r0 + _TP, _D_TR + hh * _HD:_D_TR + (hh + 1) * _HD]
                v = y[r0:r0 + _TP,
                      2 * _D_TR + hh * _HD:2 * _D_TR + (hh + 1) * _HD]
                s = lax.dot_general(q, k, (((1,), (1,)), ((), ())),
                                    preferred_element_type=jnp.float32)
                s = s * (1.0 / (_HD ** 0.5)) + keymask
                smax = jnp.max(s, axis=-1, keepdims=True)
                e = jnp.exp(s - smax)
                p = e / jnp.sum(e, axis=-1, keepdims=True)
                heads.append(lax.dot_general(p, v, (((1,), (0,)), ((), ())),
                                             preferred_element_type=jnp.float32))
            brows.append(jnp.concatenate(heads, axis=1))
        att = jnp.concatenate(brows, axis=0)
        x_ref[...] = x + jnp.dot(att, pw_ref[0],
                                 preferred_element_type=jnp.float32) + pb_ref[0]
        acc_ref[...] = jnp.zeros((_B * _TP, _D_TR), jnp.float32)

    x = x_ref[...]
    h2 = _ln(x, l2g_ref[0], l2b_ref[0]).astype(jnp.bfloat16)
    h2 = jax.nn.gelu(jnp.dot(h2, m1w_ref[0], preferred_element_type=jnp.float32)
                     + m1b_ref[0]).astype(jnp.bfloat16)
    acc_ref[...] += jnp.dot(h2, m2w_ref[0], preferred_element_type=jnp.float32)

    @pl.when(j == _KJ - 1)
    def _():
        xn = x + acc_ref[...] + m2b_ref[0]
        x_ref[...] = xn

        @pl.when(d == _DEPTH - 1)
        def _():
            cls = xn.reshape(_B, _TP, _D_TR)[:, 0, :]
            cls = _ln(cls, ng_ref[...], nb_ref[...])
            emb = jnp.dot(cls, t2ew_ref[...],
                          preferred_element_type=jnp.float32) + t2eb_ref[...]
            out_ref[...] = jnp.concatenate(
                [emb, jnp.zeros((8 - _B, _D_EMB), jnp.float32)], axis=0)


def _transformer(x0, bp, ng, nb, t2ew, t2eb):
    def lay(w):
        shape = (1,) + w.shape[1:]
        return pl.BlockSpec(shape, lambda d, j: (d,) + tuple(0 for _ in w.shape[1:]))
    const = lambda w: pl.BlockSpec(w.shape, lambda d, j: tuple(0 for _ in w.shape))
    def v3(w):
        return w.reshape(_DEPTH, 1, w.shape[1]) if w.ndim == 2 else w
    bf = lambda w: w.astype(jnp.bfloat16)
    wts = [v3(bp['ln1_g']), v3(bp['ln1_b']), bp['qkv_W'], v3(bp['qkv_b']),
           bp['proj_W'], v3(bp['proj_b']), v3(bp['ln2_g']), v3(bp['ln2_b'])]
    in_specs = [pl.BlockSpec((_B * _TP, _D_TR), lambda d, j: (0, 0))] \
        + [lay(w) for w in wts] + [
        pl.BlockSpec((1, _D_TR, _HCH), lambda d, j: (d, 0, j)),
        pl.BlockSpec((1, 1, _HCH), lambda d, j: (d, 0, j)),
        pl.BlockSpec((1, _HCH, _D_TR), lambda d, j: (d, j, 0)),
        pl.BlockSpec((1, 1, _D_TR), lambda d, j: (d, 0, 0)),
    ] + [const(w) for w in (ng, nb, t2ew, t2eb)]
    return pl.pallas_call(
        _tr_body,
        grid=(_DEPTH, _KJ),
        compiler_params=pltpu.CompilerParams(
            vmem_limit_bytes=100 * 1024 * 1024),
        in_specs=in_specs,
        out_specs=pl.BlockSpec((8, _D_EMB), lambda d, j: (0, 0)),
        out_shape=jax.ShapeDtypeStruct((8, _D_EMB), jnp.float32),
        scratch_shapes=[pltpu.VMEM((_B * _TP, _D_TR), jnp.float32),
                        pltpu.VMEM((_B * _TP, _D_TR), jnp.float32)],
    )(x0, *wts, bf(bp['mlp_W1']), v3(bp['mlp_b1']), bf(bp['mlp_W2']),
      v3(bp['mlp_b2']), ng, nb, t2ew, t2eb)


def kernel(pts, colors, params):
    p = params
    ptsT = jnp.concatenate(
        [jnp.transpose(pts, (0, 2, 1)),
         jnp.zeros((_B, 5, _N), jnp.float32)], axis=1)
    px = pts[:, :, 0].reshape(_B, _NR, _NL)
    py = pts[:, :, 1].reshape(_B, _NR, _NL)
    pz = pts[:, :, 2].reshape(_B, _NR, _NL)
    cen8 = _fps(px, py, pz)
    idxp = _knn(ptsT, cen8)

    idxf = idxp[:, :, :_M].reshape(_B * _G * _M)
    table = jnp.concatenate([pts, colors], axis=-1).reshape(_B * _N, 6)
    table = jnp.pad(table, ((0, 0), (0, 10)))
    gathered = _sc_gather(table, idxf)

    cen16 = jnp.pad(cen8.reshape(_B * _G, 8), ((0, 0), (0, 8)))
    w1p = jnp.pad(p['enc_W1'], ((0, 10), (0, 0)))
    pw1p = jnp.pad(p['pos_W1'], ((0, 13), (0, 0)))
    wts = [w1p, p['enc_b1'], p['bn1_g'], p['bn1_b'],
           p['enc_W2'], p['enc_b2'], p['enc_W3'], p['enc_b3'],
           p['bn2_g'], p['bn2_b'], p['enc_W4'], p['enc_b4'],
           p['e2t_W'], p['e2t_b'], pw1p, p['pos_b1'],
           p['pos_W2'], p['pos_b2']]
    tokpos = _encoder(gathered, cen16, wts)

    cls = jnp.broadcast_to(p['cls_token'] + p['cls_pos'], (_B, 1, _D_TR))
    x0 = jnp.concatenate([cls, tokpos.reshape(_B, _G, _D_TR)], axis=1)
    x0 = jnp.pad(x0, ((0, 0), (0, _TP - _T), (0, 0))).reshape(_B * _TP, _D_TR)
    emb = _transformer(x0, p['blocks'], p['norm_g'], p['norm_b'],
                       p['t2e_W'], p['t2e_b'])
    return emb[:_B]


# speedup vs baseline: 1.0958x; 1.0958x over previous
import functools

import jax
import jax.numpy as jnp
from jax import lax
from jax.experimental import pallas as pl
from jax.experimental.pallas import tpu as pltpu
from jax.experimental.pallas import tpu_sc as plsc

_B, _N, _G, _M = 4, 8192, 256, 32
_D_ENC, _D_TR, _D_EMB = 512, 768, 512
_DEPTH, _HEADS = 12, 12
_HD = _D_TR // _HEADS
_T = _G + 1
_TP = 264
_NR, _NL = 64, 128
_NEG = -1e30
_INF = 1e30


def _fps_body(px_ref, py_ref, pz_ref, cen_ref):
    px = px_ref[...]
    py = py_ref[...]
    pz = pz_ref[...]
    rows8 = lax.broadcasted_iota(jnp.int32, (_B, _G, 8), 1)
    cols8 = lax.broadcasted_iota(jnp.int32, (_B, _G, 8), 2)

    def red(a, op):
        return op(op(a, axis=2, keepdims=True), axis=1, keepdims=True)

    def cen_row(cx, cy, cz):
        return jnp.where(cols8 == 0, cx,
               jnp.where(cols8 == 1, cy,
               jnp.where(cols8 == 2, cz, 0.0)))

    c0x = px[:, 0:1, 0:1]
    c0y = py[:, 0:1, 0:1]
    c0z = pz[:, 0:1, 0:1]
    dists0 = (px - c0x) ** 2 + (py - c0y) ** 2 + (pz - c0z) ** 2
    cen0 = jnp.where(rows8 == 0, cen_row(c0x, c0y, c0z),
                     jnp.zeros((_B, _G, 8), jnp.float32))

    def fps_step(i, carry):
        dists, cen = carry
        m = red(dists, jnp.max)
        onehot = dists == m
        cx = red(jnp.where(onehot, px, 0.0), jnp.sum)
        cy = red(jnp.where(onehot, py, 0.0), jnp.sum)
        cz = red(jnp.where(onehot, pz, 0.0), jnp.sum)
        cen = jnp.where(rows8 == i, cen_row(cx, cy, cz), cen)
        dnew = (px - cx) ** 2 + (py - cy) ** 2 + (pz - cz) ** 2
        return jnp.minimum(dists, dnew), cen

    _, cen = lax.fori_loop(1, _G, fps_step, (dists0, cen0))
    cen_ref[...] = cen


def _fps(px, py, pz):
    full = lambda s: pl.BlockSpec(s, lambda: tuple(0 for _ in s))
    return pl.pallas_call(
        _fps_body,
        in_specs=[full((_B, _NR, _NL))] * 3,
        out_specs=full((_B, _G, 8)),
        out_shape=jax.ShapeDtypeStruct((_B, _G, 8), jnp.float32),
    )(px, py, pz)


def _knn_body(ptsT_ref, cen_ref, idx_ref):
    b = pl.program_id(0)
    cenT = cen_ref[0]
    P8 = ptsT_ref[0]
    pn = jnp.sum(P8 * P8, axis=0, keepdims=True)
    cn = jnp.sum(cenT * cenT, axis=1, keepdims=True)
    cp = lax.dot_general(cenT, P8, (((1,), (0,)), ((), ())),
                         preferred_element_type=jnp.float32)
    D = pn + cn - 2.0 * cp

    coli = lax.broadcasted_iota(jnp.int32, (_G, _N), 1)
    lane = lax.broadcasted_iota(jnp.int32, (_G, _NL), 1)
    off = b * _N

    K = lax.bitcast_convert_type(D + 1.0, jnp.int32)
    maxi = jnp.int32(0x7FFFFFFF)

    def knn_step(k, carry):
        lastk, idxbuf = carry
        m = jnp.min(jnp.where(K > lastk, K, maxi),
                    axis=1, keepdims=True)
        rowidx = jnp.min(jnp.where(K == m, coli, _N),
                         axis=1, keepdims=True)
        idxbuf = jnp.where(lane == k, rowidx + off, idxbuf)
        return m, idxbuf

    _, idxbuf = lax.fori_loop(
        0, _M, knn_step,
        (jnp.full((_G, 1), jnp.iinfo(jnp.int32).min, jnp.int32),
         jnp.zeros((_G, _NL), jnp.int32)))
    idx_ref[0] = idxbuf


def _knn(ptsT, cen8):
    return pl.pallas_call(
        _knn_body,
        grid=(_B,),
        in_specs=[
            pl.BlockSpec((1, 8, _N), lambda b: (b, 0, 0)),
            pl.BlockSpec((1, _G, 8), lambda b: (b, 0, 0)),
        ],
        out_specs=pl.BlockSpec((1, _G, _NL), lambda b: (b, 0, 0)),
        out_shape=jax.ShapeDtypeStruct((_B, _G, _NL), jnp.int32),
    )(ptsT, cen8)


def _sc_gather(table, idx):
    tot = idx.shape[0]
    info = plsc.get_sparse_core_info()
    nc = info.num_cores
    nw = nc * info.num_subcores
    bpw = tot // nw
    chunks = bpw // 128
    idx3 = idx.reshape(nw, chunks, 128)
    mesh = plsc.VectorSubcoreMesh(core_axis_name="c", subcore_axis_name="s")

    @functools.partial(
        pl.kernel,
        mesh=mesh,
        compiler_params=pltpu.CompilerParams(use_tc_tiling_on_sc=False),
        out_type=jax.ShapeDtypeStruct((tot, 16), jnp.float32),
        scratch_types=[
            pltpu.VMEM((chunks, 128), jnp.int32),
            pltpu.VMEM((bpw, 16), jnp.float32),
            pltpu.SemaphoreType.DMA,
        ],
    )
    def k(table_hbm, idx_hbm, out_hbm, idx_v, rows_v, sem):
        wid = lax.axis_index("s") * nc + lax.axis_index("c")
        pltpu.sync_copy(idx_hbm.at[wid], idx_v)
        copies = []
        for j in range(chunks):
            copies.append(pltpu.async_copy(
                table_hbm.at[idx_v.at[j]],
                rows_v.at[pl.ds(j * 128, 128)], sem))
        for c in copies:
            c.wait()
        pltpu.sync_copy(rows_v, out_hbm.at[pl.ds(wid * bpw, bpw)])

    return k(table, idx3)


_GB = 128
_ROWS = _GB * _M


def _enc_body(feat_ref, cen_ref, w1_ref, b1_ref, g1_ref, bb1_ref,
              w2_ref, b2_ref, w3_ref, b3_ref, g2_ref, bb2_ref,
              w4_ref, b4_ref, e2tw_ref, e2tb_ref,
              pw1_ref, pb1_ref, pw2_ref, pb2_ref, out_ref):
    x = feat_ref[...]
    c = cen_ref[...]
    x = (x.reshape(_GB, _M, 16) - c[:, None, :]).reshape(_ROWS, 16)
    f = jnp.dot(x, w1_ref[...], preferred_element_type=jnp.float32) + b1_ref[...]
    f = jax.nn.relu(f * g1_ref[...] + bb1_ref[...])
    f = jnp.dot(f, w2_ref[...], preferred_element_type=jnp.float32) + b2_ref[...]
    fg = jnp.max(f.reshape(_GB, _M, 256), axis=1)
    fgb = jnp.broadcast_to(fg[:, None, :], (_GB, _M, 256)).reshape(_ROWS, 256)
    f = jnp.concatenate([fgb, f], axis=-1)
    f = jnp.dot(f, w3_ref[...], preferred_element_type=jnp.float32) + b3_ref[...]
    f = jax.nn.relu(f * g2_ref[...] + bb2_ref[...])
    f = jnp.dot(f, w4_ref[...], preferred_element_type=jnp.float32) + b4_ref[...]
    tok = jnp.max(f.reshape(_GB, _M, _D_ENC), axis=1)
    tok = jnp.dot(tok, e2tw_ref[...], preferred_element_type=jnp.float32) + e2tb_ref[...]
    ph = jax.nn.gelu(jnp.dot(c, pw1_ref[...], preferred_element_type=jnp.float32)
                     + pb1_ref[...])
    pos = jnp.dot(ph, pw2_ref[...], preferred_element_type=jnp.float32) + pb2_ref[...]
    out_ref[...] = tok + pos


def _encoder(feats, cen16, wts):
    nsteps = (_B * _G) // _GB
    const = lambda shape: pl.BlockSpec(shape, lambda i: tuple(0 for _ in shape))
    in_specs = [
        pl.BlockSpec((_ROWS, 16), lambda i: (i, 0)),
        pl.BlockSpec((_GB, 16), lambda i: (i, 0)),
    ] + [const(w.shape) for w in wts]
    return pl.pallas_call(
        _enc_body,
        grid=(nsteps,),
        in_specs=in_specs,
        out_specs=pl.BlockSpec((_GB, _D_TR), lambda i: (i, 0)),
        out_shape=jax.ShapeDtypeStruct((_B * _G, _D_TR), jnp.float32),
    )(feats, cen16, *wts)


def _ln(x, g, b):
    m = jnp.mean(x, axis=-1, keepdims=True)
    v = jnp.mean((x - m) ** 2, axis=-1, keepdims=True)
    return (x - m) / jnp.sqrt(v + 1e-5) * g + b


_KJ = 4
_HCH = 4 * _D_TR // _KJ


def _tr_body(x0_ref, l1g_ref, l1b_ref, qkvw_ref, qkvb_ref, pw_ref, pb_ref,
             l2g_ref, l2b_ref, m1w_ref, m1b_ref, m2w_ref, m2b_ref,
             ng_ref, nb_ref, t2ew_ref, t2eb_ref, out_ref, x_ref, acc_ref):
    d = pl.program_id(0)
    j = pl.program_id(1)

    @pl.when((d == 0) & (j == 0))
    def _():
        x_ref[...] = x0_ref[...]

    @pl.when(j == 0)
    def _():
        x = x_ref[...]
        keymask = jnp.where(
            lax.broadcasted_iota(jnp.int32, (1, _TP), 1) >= _T, _NEG, 0.0)
        h = _ln(x, l1g_ref[0], l1b_ref[0])
        y = jnp.dot(h, qkvw_ref[0], preferred_element_type=jnp.float32) \
            + qkvb_ref[0]
        brows = []
        for b in range(_B):
            r0 = b * _TP
            heads = []
            for hh in range(_HEADS):
                q = y[r0:r0 + _TP, hh * _HD:(hh + 1) * _HD]
                k = y[r0:r0 + _TP, _D_TR + hh * _HD:_D_TR + (hh + 1) * _HD]
                v = y[r0:r0 + _TP,
                      2 * _D_TR + hh * _HD:2 * _D_TR + (hh + 1) * _HD]
                s = lax.dot_general(q, k, (((1,), (1,)), ((), ())),
                                    preferred_element_type=jnp.float32)
                s = s * (1.0 / (_HD ** 0.5)) + keymask
                smax = jnp.max(s, axis=-1, keepdims=True)
                e = jnp.exp(s - smax)
                p = e / jnp.sum(e, axis=-1, keepdims=True)
                heads.append(lax.dot_general(p, v, (((1,), (0,)), ((), ())),
                                             preferred_element_type=jnp.float32))
            brows.append(jnp.concatenate(heads, axis=1))
        att = jnp.concatenate(brows, axis=0)
        x_ref[...] = x + jnp.dot(att, pw_ref[0],
                                 preferred_element_type=jnp.float32) + pb_ref[0]
        acc_ref[...] = jnp.zeros((_B * _TP, _D_TR), jnp.float32)

    x = x_ref[...]
    h2 = _ln(x, l2g_ref[0], l2b_ref[0])
    h2 = jax.nn.gelu(jnp.dot(h2, m1w_ref[0], preferred_element_type=jnp.float32)
                     + m1b_ref[0])
    acc_ref[...] += jnp.dot(h2, m2w_ref[0], preferred_element_type=jnp.float32)

    @pl.when(j == _KJ - 1)
    def _():
        xn = x + acc_ref[...] + m2b_ref[0]
        x_ref[...] = xn

        @pl.when(d == _DEPTH - 1)
        def _():
            cls = xn.reshape(_B, _TP, _D_TR)[:, 0, :]
            cls = _ln(cls, ng_ref[...], nb_ref[...])
            emb = jnp.dot(cls, t2ew_ref[...],
                          preferred_element_type=jnp.float32) + t2eb_ref[...]
            out_ref[...] = jnp.concatenate(
                [emb, jnp.zeros((8 - _B, _D_EMB), jnp.float32)], axis=0)


def _transformer(x0, bp, ng, nb, t2ew, t2eb):
    def lay(w):
        shape = (1,) + w.shape[1:]
        return pl.BlockSpec(shape, lambda d, j: (d,) + tuple(0 for _ in w.shape[1:]))
    const = lambda w: pl.BlockSpec(w.shape, lambda d, j: tuple(0 for _ in w.shape))
    def v3(w):
        return w.reshape(_DEPTH, 1, w.shape[1]) if w.ndim == 2 else w
    wts = [v3(bp['ln1_g']), v3(bp['ln1_b']), bp['qkv_W'], v3(bp['qkv_b']),
           bp['proj_W'], v3(bp['proj_b']), v3(bp['ln2_g']), v3(bp['ln2_b'])]
    in_specs = [pl.BlockSpec((_B * _TP, _D_TR), lambda d, j: (0, 0))] \
        + [lay(w) for w in wts] + [
        pl.BlockSpec((1, _D_TR, _HCH), lambda d, j: (d, 0, j)),
        pl.BlockSpec((1, 1, _HCH), lambda d, j: (d, 0, j)),
        pl.BlockSpec((1, _HCH, _D_TR), lambda d, j: (d, j, 0)),
        pl.BlockSpec((1, 1, _D_TR), lambda d, j: (d, 0, 0)),
    ] + [const(w) for w in (ng, nb, t2ew, t2eb)]
    return pl.pallas_call(
        _tr_body,
        grid=(_DEPTH, _KJ),
        compiler_params=pltpu.CompilerParams(
            vmem_limit_bytes=100 * 1024 * 1024),
        in_specs=in_specs,
        out_specs=pl.BlockSpec((8, _D_EMB), lambda d, j: (0, 0)),
        out_shape=jax.ShapeDtypeStruct((8, _D_EMB), jnp.float32),
        scratch_shapes=[pltpu.VMEM((_B * _TP, _D_TR), jnp.float32),
                        pltpu.VMEM((_B * _TP, _D_TR), jnp.float32)],
    )(x0, *wts, bp['mlp_W1'], v3(bp['mlp_b1']), bp['mlp_W2'],
      v3(bp['mlp_b2']), ng, nb, t2ew, t2eb)


def kernel(pts, colors, params):
    p = params
    ptsT = jnp.concatenate(
        [jnp.transpose(pts, (0, 2, 1)),
         jnp.zeros((_B, 5, _N), jnp.float32)], axis=1)
    px = pts[:, :, 0].reshape(_B, _NR, _NL)
    py = pts[:, :, 1].reshape(_B, _NR, _NL)
    pz = pts[:, :, 2].reshape(_B, _NR, _NL)
    cen8 = _fps(px, py, pz)
    idxp = _knn(ptsT, cen8)

    idxf = idxp[:, :, :_M].reshape(_B * _G * _M)
    table = jnp.concatenate([pts, colors], axis=-1).reshape(_B * _N, 6)
    table = jnp.pad(table, ((0, 0), (0, 10)))
    gathered = _sc_gather(table, idxf)

    cen16 = jnp.pad(cen8.reshape(_B * _G, 8), ((0, 0), (0, 8)))
    w1p = jnp.pad(p['enc_W1'], ((0, 10), (0, 0)))
    pw1p = jnp.pad(p['pos_W1'], ((0, 13), (0, 0)))
    wts = [w1p, p['enc_b1'], p['bn1_g'], p['bn1_b'],
           p['enc_W2'], p['enc_b2'], p['enc_W3'], p['enc_b3'],
           p['bn2_g'], p['bn2_b'], p['enc_W4'], p['enc_b4'],
           p['e2t_W'], p['e2t_b'], pw1p, p['pos_b1'],
           p['pos_W2'], p['pos_b2']]
    tokpos = _encoder(gathered, cen16, wts)

    cls = jnp.broadcast_to(p['cls_token'] + p['cls_pos'], (_B, 1, _D_TR))
    x0 = jnp.concatenate([cls, tokpos.reshape(_B, _G, _D_TR)], axis=1)
    x0 = jnp.pad(x0, ((0, 0), (0, _TP - _T), (0, 0))).reshape(_B * _TP, _D_TR)
    emb = _transformer(x0, p['blocks'], p['norm_g'], p['norm_b'],
                       p['t2e_W'], p['t2e_b'])
    return emb[:_B]


# speedup vs baseline: 1.1506x; 1.0500x over previous
import functools

import jax
import jax.numpy as jnp
from jax import lax
from jax.experimental import pallas as pl
from jax.experimental.pallas import tpu as pltpu
from jax.experimental.pallas import tpu_sc as plsc

_B, _N, _G, _M = 4, 8192, 256, 32
_D_ENC, _D_TR, _D_EMB = 512, 768, 512
_DEPTH, _HEADS = 12, 12
_HD = _D_TR // _HEADS
_T = _G + 1
_TP = 264
_NR, _NL = 64, 128
_NEG = -1e30
_INF = 1e30


def _fps_body(px_ref, py_ref, pz_ref, cen_ref):
    px = px_ref[...]
    py = py_ref[...]
    pz = pz_ref[...]
    rows8 = lax.broadcasted_iota(jnp.int32, (_B, _G, 8), 1)
    cols8 = lax.broadcasted_iota(jnp.int32, (_B, _G, 8), 2)

    def red(a, op):
        return op(op(a, axis=2, keepdims=True), axis=1, keepdims=True)

    def cen_row(cx, cy, cz):
        return jnp.where(cols8 == 0, cx,
               jnp.where(cols8 == 1, cy,
               jnp.where(cols8 == 2, cz, 0.0)))

    c0x = px[:, 0:1, 0:1]
    c0y = py[:, 0:1, 0:1]
    c0z = pz[:, 0:1, 0:1]
    dists0 = (px - c0x) ** 2 + (py - c0y) ** 2 + (pz - c0z) ** 2
    cen0 = jnp.where(rows8 == 0, cen_row(c0x, c0y, c0z),
                     jnp.zeros((_B, _G, 8), jnp.float32))

    def fps_step(i, carry):
        dists, cen = carry
        m = red(dists, jnp.max)
        onehot = dists == m
        cx = red(jnp.where(onehot, px, 0.0), jnp.sum)
        cy = red(jnp.where(onehot, py, 0.0), jnp.sum)
        cz = red(jnp.where(onehot, pz, 0.0), jnp.sum)
        cen = jnp.where(rows8 == i, cen_row(cx, cy, cz), cen)
        dnew = (px - cx) ** 2 + (py - cy) ** 2 + (pz - cz) ** 2
        return jnp.minimum(dists, dnew), cen

    _, cen = lax.fori_loop(1, _G, fps_step, (dists0, cen0))
    cen_ref[...] = cen


def _fps(px, py, pz):
    full = lambda s: pl.BlockSpec(s, lambda: tuple(0 for _ in s))
    return pl.pallas_call(
        _fps_body,
        in_specs=[full((_B, _NR, _NL))] * 3,
        out_specs=full((_B, _G, 8)),
        out_shape=jax.ShapeDtypeStruct((_B, _G, 8), jnp.float32),
    )(px, py, pz)


def _knn_body(ptsT_ref, cen_ref, idx_ref):
    b = pl.program_id(0)
    cenT = cen_ref[0]
    P8 = ptsT_ref[0]
    pn = jnp.sum(P8 * P8, axis=0, keepdims=True)
    cn = jnp.sum(cenT * cenT, axis=1, keepdims=True)
    cp = lax.dot_general(cenT, P8, (((1,), (0,)), ((), ())),
                         preferred_element_type=jnp.float32)
    D = pn + cn - 2.0 * cp

    coli = lax.broadcasted_iota(jnp.int32, (_G, _N), 1)
    lane = lax.broadcasted_iota(jnp.int32, (_G, _NL), 1)
    off = b * _N

    def knn_step(k, carry):
        lastd, idxbuf = carry
        m = jnp.min(jnp.where(D > lastd, D, _INF),
                    axis=1, keepdims=True)
        rowidx = jnp.min(jnp.where(D == m, coli, _N),
                         axis=1, keepdims=True)
        idxbuf = jnp.where(lane == k, rowidx + off, idxbuf)
        return m, idxbuf

    _, idxbuf = lax.fori_loop(
        0, _M, knn_step,
        (jnp.full((_G, 1), -_INF, jnp.float32),
         jnp.zeros((_G, _NL), jnp.int32)))
    idx_ref[0] = idxbuf


def _knn(ptsT, cen8):
    return pl.pallas_call(
        _knn_body,
        grid=(_B,),
        in_specs=[
            pl.BlockSpec((1, 8, _N), lambda b: (b, 0, 0)),
            pl.BlockSpec((1, _G, 8), lambda b: (b, 0, 0)),
        ],
        out_specs=pl.BlockSpec((1, _G, _NL), lambda b: (b, 0, 0)),
        out_shape=jax.ShapeDtypeStruct((_B, _G, _NL), jnp.int32),
    )(ptsT, cen8)


def _sc_gather(table, idx):
    tot = idx.shape[0]
    info = plsc.get_sparse_core_info()
    nc = info.num_cores
    nw = nc * info.num_subcores
    bpw = tot // nw
    chunks = bpw // 128
    idx3 = idx.reshape(nw, chunks, 128)
    mesh = plsc.VectorSubcoreMesh(core_axis_name="c", subcore_axis_name="s")

    @functools.partial(
        pl.kernel,
        mesh=mesh,
        compiler_params=pltpu.CompilerParams(use_tc_tiling_on_sc=False),
        out_type=jax.ShapeDtypeStruct((tot, 16), jnp.float32),
        scratch_types=[
            pltpu.VMEM((chunks, 128), jnp.int32),
            pltpu.VMEM((bpw, 16), jnp.float32),
            pltpu.SemaphoreType.DMA,
        ],
    )
    def k(table_hbm, idx_hbm, out_hbm, idx_v, rows_v, sem):
        wid = lax.axis_index("s") * nc + lax.axis_index("c")
        pltpu.sync_copy(idx_hbm.at[wid], idx_v)
        copies = []
        for j in range(chunks):
            copies.append(pltpu.async_copy(
                table_hbm.at[idx_v.at[j]],
                rows_v.at[pl.ds(j * 128, 128)], sem))
        for c in copies:
            c.wait()
        pltpu.sync_copy(rows_v, out_hbm.at[pl.ds(wid * bpw, bpw)])

    return k(table, idx3)


_GB = 128
_ROWS = _GB * _M


def _enc_body(feat_ref, cen_ref, w1_ref, b1_ref, g1_ref, bb1_ref,
              w2_ref, b2_ref, w3_ref, b3_ref, g2_ref, bb2_ref,
              w4_ref, b4_ref, e2tw_ref, e2tb_ref,
              pw1_ref, pb1_ref, pw2_ref, pb2_ref, out_ref):
    x = feat_ref[...]
    c = cen_ref[...]
    x = (x.reshape(_GB, _M, 16) - c[:, None, :]).reshape(_ROWS, 16)
    f = jnp.dot(x, w1_ref[...], preferred_element_type=jnp.float32) + b1_ref[...]
    f = jax.nn.relu(f * g1_ref[...] + bb1_ref[...])
    f = jnp.dot(f, w2_ref[...], preferred_element_type=jnp.float32) + b2_ref[...]
    fg = jnp.max(f.reshape(_GB, _M, 256), axis=1)
    fgb = jnp.broadcast_to(fg[:, None, :], (_GB, _M, 256)).reshape(_ROWS, 256)
    f = jnp.concatenate([fgb, f], axis=-1)
    f = jnp.dot(f, w3_ref[...], preferred_element_type=jnp.float32) + b3_ref[...]
    f = jax.nn.relu(f * g2_ref[...] + bb2_ref[...])
    f = jnp.dot(f, w4_ref[...], preferred_element_type=jnp.float32) + b4_ref[...]
    tok = jnp.max(f.reshape(_GB, _M, _D_ENC), axis=1)
    tok = jnp.dot(tok, e2tw_ref[...], preferred_element_type=jnp.float32) + e2tb_ref[...]
    ph = jax.nn.gelu(jnp.dot(c, pw1_ref[...], preferred_element_type=jnp.float32)
                     + pb1_ref[...])
    pos = jnp.dot(ph, pw2_ref[...], preferred_element_type=jnp.float32) + pb2_ref[...]
    out_ref[...] = tok + pos


def _encoder(feats, cen16, wts):
    nsteps = (_B * _G) // _GB
    const = lambda shape: pl.BlockSpec(shape, lambda i: tuple(0 for _ in shape))
    in_specs = [
        pl.BlockSpec((_ROWS, 16), lambda i: (i, 0)),
        pl.BlockSpec((_GB, 16), lambda i: (i, 0)),
    ] + [const(w.shape) for w in wts]
    return pl.pallas_call(
        _enc_body,
        grid=(nsteps,),
        in_specs=in_specs,
        out_specs=pl.BlockSpec((_GB, _D_TR), lambda i: (i, 0)),
        out_shape=jax.ShapeDtypeStruct((_B * _G, _D_TR), jnp.float32),
    )(feats, cen16, *wts)


def _ln(x, g, b):
    m = jnp.mean(x, axis=-1, keepdims=True)
    v = jnp.mean((x - m) ** 2, axis=-1, keepdims=True)
    return (x - m) / jnp.sqrt(v + 1e-5) * g + b


_KJ = 4
_HCH = 4 * _D_TR // _KJ


def _tr_body(x0_ref, l1g_ref, l1b_ref, qkvw_ref, qkvb_ref, pw_ref, pb_ref,
             l2g_ref, l2b_ref, m1w_ref, m1b_ref, m2w_ref, m2b_ref,
             ng_ref, nb_ref, t2ew_ref, t2eb_ref, out_ref, x_ref, acc_ref):
    d = pl.program_id(0)
    j = pl.program_id(1)

    @pl.when((d == 0) & (j == 0))
    def _():
        x_ref[...] = x0_ref[...]

    @pl.when(j == 0)
    def _():
        x = x_ref[...]
        keymask = jnp.where(
            lax.broadcasted_iota(jnp.int32, (1, _TP), 1) >= _T, _NEG, 0.0)
        h = _ln(x, l1g_ref[0], l1b_ref[0])
        y = jnp.dot(h, qkvw_ref[0], preferred_element_type=jnp.float32) \
            + qkvb_ref[0]
        brows = []
        for b in range(_B):
            r0 = b * _TP
            heads = []
            for hh in range(_HEADS):
                q = y[r0:r0 + _TP, hh * _HD:(hh + 1) * _HD]
                k = y[r0:r0 + _TP, _D_TR + hh * _HD:_D_TR + (hh + 1) * _HD]
                v = y[r0:r0 + _TP,
                      2 * _D_TR + hh * _HD:2 * _D_TR + (hh + 1) * _HD]
                s = lax.dot_general(q, k, (((1,), (1,)), ((), ())),
                                    preferred_element_type=jnp.float32)
                s = s * (1.0 / (_HD ** 0.5)) + keymask
                smax = jnp.max(s, axis=-1, keepdims=True)
                e = jnp.exp(s - smax)
                p = e / jnp.sum(e, axis=-1, keepdims=True)
                heads.append(lax.dot_general(p, v, (((1,), (0,)), ((), ())),
                                             preferred_element_type=jnp.float32))
            brows.append(jnp.concatenate(heads, axis=1))
        att = jnp.concatenate(brows, axis=0)
        x_ref[...] = x + jnp.dot(att, pw_ref[0],
                                 preferred_element_type=jnp.float32) + pb_ref[0]
        acc_ref[...] = jnp.zeros((_B * _TP, _D_TR), jnp.float32)

    x = x_ref[...]
    h2 = _ln(x, l2g_ref[0], l2b_ref[0])
    h2 = jax.nn.gelu(jnp.dot(h2, m1w_ref[0], preferred_element_type=jnp.float32)
                     + m1b_ref[0])
    acc_ref[...] += jnp.dot(h2, m2w_ref[0], preferred_element_type=jnp.float32)

    @pl.when(j == _KJ - 1)
    def _():
        xn = x + acc_ref[...] + m2b_ref[0]
        x_ref[...] = xn

        @pl.when(d == _DEPTH - 1)
        def _():
            cls = xn.reshape(_B, _TP, _D_TR)[:, 0, :]
            cls = _ln(cls, ng_ref[...], nb_ref[...])
            emb = jnp.dot(cls, t2ew_ref[...],
                          preferred_element_type=jnp.float32) + t2eb_ref[...]
            out_ref[...] = jnp.concatenate(
                [emb, jnp.zeros((8 - _B, _D_EMB), jnp.float32)], axis=0)


def _transformer(x0, bp, ng, nb, t2ew, t2eb):
    def lay(w):
        shape = (1,) + w.shape[1:]
        return pl.BlockSpec(shape, lambda d, j: (d,) + tuple(0 for _ in w.shape[1:]))
    const = lambda w: pl.BlockSpec(w.shape, lambda d, j: tuple(0 for _ in w.shape))
    def v3(w):
        return w.reshape(_DEPTH, 1, w.shape[1]) if w.ndim == 2 else w
    wts = [v3(bp['ln1_g']), v3(bp['ln1_b']), bp['qkv_W'], v3(bp['qkv_b']),
           bp['proj_W'], v3(bp['proj_b']), v3(bp['ln2_g']), v3(bp['ln2_b'])]
    in_specs = [pl.BlockSpec((_B * _TP, _D_TR), lambda d, j: (0, 0))] \
        + [lay(w) for w in wts] + [
        pl.BlockSpec((1, _D_TR, _HCH), lambda d, j: (d, 0, j)),
        pl.BlockSpec((1, 1, _HCH), lambda d, j: (d, 0, j)),
        pl.BlockSpec((1, _HCH, _D_TR), lambda d, j: (d, j, 0)),
        pl.BlockSpec((1, 1, _D_TR), lambda d, j: (d, 0, 0)),
    ] + [const(w) for w in (ng, nb, t2ew, t2eb)]
    return pl.pallas_call(
        _tr_body,
        grid=(_DEPTH, _KJ),
        compiler_params=pltpu.CompilerParams(
            vmem_limit_bytes=100 * 1024 * 1024),
        in_specs=in_specs,
        out_specs=pl.BlockSpec((8, _D_EMB), lambda d, j: (0, 0)),
        out_shape=jax.ShapeDtypeStruct((8, _D_EMB), jnp.float32),
        scratch_shapes=[pltpu.VMEM((_B * _TP, _D_TR), jnp.float32),
                        pltpu.VMEM((_B * _TP, _D_TR), jnp.float32)],
    )(x0, *wts, bp['mlp_W1'], v3(bp['mlp_b1']), bp['mlp_W2'],
      v3(bp['mlp_b2']), ng, nb, t2ew, t2eb)


def kernel(pts, colors, params):
    p = params
    ptsT = jnp.concatenate(
        [jnp.transpose(pts, (0, 2, 1)),
         jnp.zeros((_B, 5, _N), jnp.float32)], axis=1)
    px = pts[:, :, 0].reshape(_B, _NR, _NL)
    py = pts[:, :, 1].reshape(_B, _NR, _NL)
    pz = pts[:, :, 2].reshape(_B, _NR, _NL)
    cen8 = _fps(px, py, pz)
    idxp = _knn(ptsT, cen8)

    idxf = idxp[:, :, :_M].reshape(_B * _G * _M)
    table = jnp.concatenate([pts, colors], axis=-1).reshape(_B * _N, 6)
    table = jnp.pad(table, ((0, 0), (0, 10)))
    gathered = _sc_gather(table, idxf)

    cen16 = jnp.pad(cen8.reshape(_B * _G, 8), ((0, 0), (0, 8)))
    w1p = jnp.pad(p['enc_W1'], ((0, 10), (0, 0)))
    pw1p = jnp.pad(p['pos_W1'], ((0, 13), (0, 0)))
    wts = [w1p, p['enc_b1'], p['bn1_g'], p['bn1_b'],
           p['enc_W2'], p['enc_b2'], p['enc_W3'], p['enc_b3'],
           p['bn2_g'], p['bn2_b'], p['enc_W4'], p['enc_b4'],
           p['e2t_W'], p['e2t_b'], pw1p, p['pos_b1'],
           p['pos_W2'], p['pos_b2']]
    tokpos = _encoder(gathered, cen16, wts)

    cls = jnp.broadcast_to(p['cls_token'] + p['cls_pos'], (_B, 1, _D_TR))
    x0 = jnp.concatenate([cls, tokpos.reshape(_B, _G, _D_TR)], axis=1)
    x0 = jnp.pad(x0, ((0, 0), (0, _TP - _T), (0, 0))).reshape(_B * _TP, _D_TR)
    emb = _transformer(x0, p['blocks'], p['norm_g'], p['norm_b'],
                       p['t2e_W'], p['t2e_b'])
    return emb[:_B]


# speedup vs baseline: 1.2255x; 1.0651x over previous
import functools

import jax
import jax.numpy as jnp
from jax import lax
from jax.experimental import pallas as pl
from jax.experimental.pallas import tpu as pltpu
from jax.experimental.pallas import tpu_sc as plsc

_B, _N, _G, _M = 4, 8192, 256, 32
_D_ENC, _D_TR, _D_EMB = 512, 768, 512
_DEPTH, _HEADS = 12, 12
_HD = _D_TR // _HEADS
_T = _G + 1
_TP = 264
_NR, _NL = 64, 128
_NEG = -1e30
_INF = 1e30


def _fps_body(px_ref, py_ref, pz_ref, cen_ref):
    px = px_ref[...]
    py = py_ref[...]
    pz = pz_ref[...]
    rows8 = lax.broadcasted_iota(jnp.int32, (_B, _G, 8), 1)
    cols8 = lax.broadcasted_iota(jnp.int32, (_B, _G, 8), 2)

    def red(a, op):
        return op(op(a, axis=2, keepdims=True), axis=1, keepdims=True)

    def cen_row(cx, cy, cz):
        return jnp.where(cols8 == 0, cx,
               jnp.where(cols8 == 1, cy,
               jnp.where(cols8 == 2, cz, 0.0)))

    c0x = px[:, 0:1, 0:1]
    c0y = py[:, 0:1, 0:1]
    c0z = pz[:, 0:1, 0:1]
    dists0 = (px - c0x) ** 2 + (py - c0y) ** 2 + (pz - c0z) ** 2
    cen0 = jnp.where(rows8 == 0, cen_row(c0x, c0y, c0z),
                     jnp.zeros((_B, _G, 8), jnp.float32))

    def fps_step(i, carry):
        dists, cen = carry
        m = red(dists, jnp.max)
        onehot = dists == m
        cx = red(jnp.where(onehot, px, 0.0), jnp.sum)
        cy = red(jnp.where(onehot, py, 0.0), jnp.sum)
        cz = red(jnp.where(onehot, pz, 0.0), jnp.sum)
        cen = jnp.where(rows8 == i, cen_row(cx, cy, cz), cen)
        dnew = (px - cx) ** 2 + (py - cy) ** 2 + (pz - cz) ** 2
        return jnp.minimum(dists, dnew), cen

    _, cen = lax.fori_loop(1, _G, fps_step, (dists0, cen0))
    cen_ref[...] = cen


def _fps(px, py, pz):
    full = lambda s: pl.BlockSpec(s, lambda: tuple(0 for _ in s))
    return pl.pallas_call(
        _fps_body,
        in_specs=[full((_B, _NR, _NL))] * 3,
        out_specs=full((_B, _G, 8)),
        out_shape=jax.ShapeDtypeStruct((_B, _G, 8), jnp.float32),
    )(px, py, pz)


def _knn_body(ptsT_ref, cen_ref, idx_ref):
    b = pl.program_id(0)
    cenT = cen_ref[0]
    P8 = ptsT_ref[0]
    pn = jnp.sum(P8 * P8, axis=0, keepdims=True)
    cn = jnp.sum(cenT * cenT, axis=1, keepdims=True)
    cp = lax.dot_general(cenT, P8, (((1,), (0,)), ((), ())),
                         preferred_element_type=jnp.float32)
    D = pn + cn - 2.0 * cp

    coli = lax.broadcasted_iota(jnp.int32, (_G, _N), 1)
    lane = lax.broadcasted_iota(jnp.int32, (_G, _NL), 1)
    off = b * _N

    def knn_step(k, carry):
        lastd, idxbuf = carry
        m = jnp.min(jnp.where(D > lastd, D, _INF),
                    axis=1, keepdims=True)
        rowidx = jnp.min(jnp.where(D == m, coli, _N),
                         axis=1, keepdims=True)
        idxbuf = jnp.where(lane == k, rowidx + off, idxbuf)
        return m, idxbuf

    _, idxbuf = lax.fori_loop(
        0, _M, knn_step,
        (jnp.full((_G, 1), -_INF, jnp.float32),
         jnp.zeros((_G, _NL), jnp.int32)))
    idx_ref[0] = idxbuf


def _knn(ptsT, cen8):
    return pl.pallas_call(
        _knn_body,
        grid=(_B,),
        in_specs=[
            pl.BlockSpec((1, 8, _N), lambda b: (b, 0, 0)),
            pl.BlockSpec((1, _G, 8), lambda b: (b, 0, 0)),
        ],
        out_specs=pl.BlockSpec((1, _G, _NL), lambda b: (b, 0, 0)),
        out_shape=jax.ShapeDtypeStruct((_B, _G, _NL), jnp.int32),
    )(ptsT, cen8)


def _sc_gather(table, idx):
    tot = idx.shape[0]
    info = plsc.get_sparse_core_info()
    nc = info.num_cores
    nw = nc * info.num_subcores
    bpw = tot // nw
    chunks = bpw // 128
    idx3 = idx.reshape(nw, chunks, 128)
    mesh = plsc.VectorSubcoreMesh(core_axis_name="c", subcore_axis_name="s")

    @functools.partial(
        pl.kernel,
        mesh=mesh,
        compiler_params=pltpu.CompilerParams(use_tc_tiling_on_sc=False),
        out_type=jax.ShapeDtypeStruct((tot, 16), jnp.float32),
        scratch_types=[
            pltpu.VMEM((chunks, 128), jnp.int32),
            pltpu.VMEM((bpw, 16), jnp.float32),
            pltpu.SemaphoreType.DMA,
        ],
    )
    def k(table_hbm, idx_hbm, out_hbm, idx_v, rows_v, sem):
        wid = lax.axis_index("s") * nc + lax.axis_index("c")
        pltpu.sync_copy(idx_hbm.at[wid], idx_v)
        copies = []
        for j in range(chunks):
            copies.append(pltpu.async_copy(
                table_hbm.at[idx_v.at[j]],
                rows_v.at[pl.ds(j * 128, 128)], sem))
        for c in copies:
            c.wait()
        pltpu.sync_copy(rows_v, out_hbm.at[pl.ds(wid * bpw, bpw)])

    return k(table, idx3)


_GB = 128
_ROWS = _GB * _M


def _enc_body(feat_ref, cen_ref, w1_ref, b1_ref, g1_ref, bb1_ref,
              w2_ref, b2_ref, w3_ref, b3_ref, g2_ref, bb2_ref,
              w4_ref, b4_ref, e2tw_ref, e2tb_ref,
              pw1_ref, pb1_ref, pw2_ref, pb2_ref, out_ref):
    x = feat_ref[...]
    c = cen_ref[...]
    x = (x.reshape(_GB, _M, 16) - c[:, None, :]).reshape(_ROWS, 16)
    f = jnp.dot(x, w1_ref[...], preferred_element_type=jnp.float32) + b1_ref[...]
    f = jax.nn.relu(f * g1_ref[...] + bb1_ref[...])
    f = jnp.dot(f, w2_ref[...], preferred_element_type=jnp.float32) + b2_ref[...]
    fg = jnp.max(f.reshape(_GB, _M, 256), axis=1)
    fgb = jnp.broadcast_to(fg[:, None, :], (_GB, _M, 256)).reshape(_ROWS, 256)
    f = jnp.concatenate([fgb, f], axis=-1)
    f = jnp.dot(f, w3_ref[...], preferred_element_type=jnp.float32) + b3_ref[...]
    f = jax.nn.relu(f * g2_ref[...] + bb2_ref[...])
    f = jnp.dot(f, w4_ref[...], preferred_element_type=jnp.float32) + b4_ref[...]
    tok = jnp.max(f.reshape(_GB, _M, _D_ENC), axis=1)
    tok = jnp.dot(tok, e2tw_ref[...], preferred_element_type=jnp.float32) + e2tb_ref[...]
    ph = jax.nn.gelu(jnp.dot(c, pw1_ref[...], preferred_element_type=jnp.float32)
                     + pb1_ref[...])
    pos = jnp.dot(ph, pw2_ref[...], preferred_element_type=jnp.float32) + pb2_ref[...]
    out_ref[...] = tok + pos


def _encoder(feats, cen16, wts):
    nsteps = (_B * _G) // _GB
    const = lambda shape: pl.BlockSpec(shape, lambda i: tuple(0 for _ in shape))
    in_specs = [
        pl.BlockSpec((_ROWS, 16), lambda i: (i, 0)),
        pl.BlockSpec((_GB, 16), lambda i: (i, 0)),
    ] + [const(w.shape) for w in wts]
    return pl.pallas_call(
        _enc_body,
        grid=(nsteps,),
        in_specs=in_specs,
        out_specs=pl.BlockSpec((_GB, _D_TR), lambda i: (i, 0)),
        out_shape=jax.ShapeDtypeStruct((_B * _G, _D_TR), jnp.float32),
    )(feats, cen16, *wts)


def _ln(x, g, b):
    m = jnp.mean(x, axis=-1, keepdims=True)
    v = jnp.mean((x - m) ** 2, axis=-1, keepdims=True)
    return (x - m) / jnp.sqrt(v + 1e-5) * g + b


_KJ = 4
_HCH = 4 * _D_TR // _KJ


def _tr_body(x0_ref, l1g_ref, l1b_ref, qkvw_ref, qkvb_ref, pw_ref, pb_ref,
             l2g_ref, l2b_ref, m1w_ref, m1b_ref, m2w_ref, m2b_ref,
             ng_ref, nb_ref, t2ew_ref, t2eb_ref, out_ref, x_ref, acc_ref):
    d = pl.program_id(0)
    j = pl.program_id(1)

    @pl.when((d == 0) & (j == 0))
    def _():
        x_ref[...] = x0_ref[...]

    @pl.when(j == 0)
    def _():
        x = x_ref[...]
        keymask = jnp.where(
            lax.broadcasted_iota(jnp.int32, (1, 1, _TP), 2) >= _T, _NEG, 0.0)
        h = _ln(x, l1g_ref[0], l1b_ref[0])
        y = jnp.dot(h, qkvw_ref[0], preferred_element_type=jnp.float32) \
            + qkvb_ref[0]
        y3 = y.reshape(_B, _TP, 3 * _D_TR)
        heads = []
        for hh in range(_HEADS):
            q = y3[:, :, hh * _HD:(hh + 1) * _HD]
            k = y3[:, :, _D_TR + hh * _HD:_D_TR + (hh + 1) * _HD]
            v = y3[:, :, 2 * _D_TR + hh * _HD:2 * _D_TR + (hh + 1) * _HD]
            s = lax.dot_general(q, k, (((2,), (2,)), ((0,), (0,))),
                                preferred_element_type=jnp.float32)
            s = s * (1.0 / (_HD ** 0.5)) + keymask
            smax = jnp.max(s, axis=-1, keepdims=True)
            e = jnp.exp(s - smax)
            p = e / jnp.sum(e, axis=-1, keepdims=True)
            heads.append(lax.dot_general(p, v, (((2,), (1,)), ((0,), (0,))),
                                         preferred_element_type=jnp.float32))
        att = jnp.concatenate(heads, axis=2).reshape(_B * _TP, _D_TR)
        x_ref[...] = x + jnp.dot(att, pw_ref[0],
                                 preferred_element_type=jnp.float32) + pb_ref[0]
        acc_ref[...] = jnp.zeros((_B * _TP, _D_TR), jnp.float32)

    x = x_ref[...]
    h2 = _ln(x, l2g_ref[0], l2b_ref[0])
    h2 = jax.nn.gelu(jnp.dot(h2, m1w_ref[0], preferred_element_type=jnp.float32)
                     + m1b_ref[0])
    acc_ref[...] += jnp.dot(h2, m2w_ref[0], preferred_element_type=jnp.float32)

    @pl.when(j == _KJ - 1)
    def _():
        xn = x + acc_ref[...] + m2b_ref[0]
        x_ref[...] = xn

        @pl.when(d == _DEPTH - 1)
        def _():
            cls = xn.reshape(_B, _TP, _D_TR)[:, 0, :]
            cls = _ln(cls, ng_ref[...], nb_ref[...])
            emb = jnp.dot(cls, t2ew_ref[...],
                          preferred_element_type=jnp.float32) + t2eb_ref[...]
            out_ref[...] = jnp.concatenate(
                [emb, jnp.zeros((8 - _B, _D_EMB), jnp.float32)], axis=0)


def _transformer(x0, bp, ng, nb, t2ew, t2eb):
    def lay(w):
        shape = (1,) + w.shape[1:]
        return pl.BlockSpec(shape, lambda d, j: (d,) + tuple(0 for _ in w.shape[1:]))
    const = lambda w: pl.BlockSpec(w.shape, lambda d, j: tuple(0 for _ in w.shape))
    def v3(w):
        return w.reshape(_DEPTH, 1, w.shape[1]) if w.ndim == 2 else w
    wts = [v3(bp['ln1_g']), v3(bp['ln1_b']), bp['qkv_W'], v3(bp['qkv_b']),
           bp['proj_W'], v3(bp['proj_b']), v3(bp['ln2_g']), v3(bp['ln2_b'])]
    in_specs = [pl.BlockSpec((_B * _TP, _D_TR), lambda d, j: (0, 0))] \
        + [lay(w) for w in wts] + [
        pl.BlockSpec((1, _D_TR, _HCH), lambda d, j: (d, 0, j)),
        pl.BlockSpec((1, 1, _HCH), lambda d, j: (d, 0, j)),
        pl.BlockSpec((1, _HCH, _D_TR), lambda d, j: (d, j, 0)),
        pl.BlockSpec((1, 1, _D_TR), lambda d, j: (d, 0, 0)),
    ] + [const(w) for w in (ng, nb, t2ew, t2eb)]
    return pl.pallas_call(
        _tr_body,
        grid=(_DEPTH, _KJ),
        compiler_params=pltpu.CompilerParams(
            vmem_limit_bytes=100 * 1024 * 1024),
        in_specs=in_specs,
        out_specs=pl.BlockSpec((8, _D_EMB), lambda d, j: (0, 0)),
        out_shape=jax.ShapeDtypeStruct((8, _D_EMB), jnp.float32),
        scratch_shapes=[pltpu.VMEM((_B * _TP, _D_TR), jnp.float32),
                        pltpu.VMEM((_B * _TP, _D_TR), jnp.float32)],
    )(x0, *wts, bp['mlp_W1'], v3(bp['mlp_b1']), bp['mlp_W2'],
      v3(bp['mlp_b2']), ng, nb, t2ew, t2eb)


def kernel(pts, colors, params):
    p = params
    ptsT = jnp.concatenate(
        [jnp.transpose(pts, (0, 2, 1)),
         jnp.zeros((_B, 5, _N), jnp.float32)], axis=1)
    px = pts[:, :, 0].reshape(_B, _NR, _NL)
    py = pts[:, :, 1].reshape(_B, _NR, _NL)
    pz = pts[:, :, 2].reshape(_B, _NR, _NL)
    cen8 = _fps(px, py, pz)
    idxp = _knn(ptsT, cen8)

    idxf = idxp[:, :, :_M].reshape(_B * _G * _M)
    table = jnp.concatenate([pts, colors], axis=-1).reshape(_B * _N, 6)
    table = jnp.pad(table, ((0, 0), (0, 10)))
    gathered = _sc_gather(table, idxf)

    cen16 = jnp.pad(cen8.reshape(_B * _G, 8), ((0, 0), (0, 8)))
    w1p = jnp.pad(p['enc_W1'], ((0, 10), (0, 0)))
    pw1p = jnp.pad(p['pos_W1'], ((0, 13), (0, 0)))
    wts = [w1p, p['enc_b1'], p['bn1_g'], p['bn1_b'],
           p['enc_W2'], p['enc_b2'], p['enc_W3'], p['enc_b3'],
           p['bn2_g'], p['bn2_b'], p['enc_W4'], p['enc_b4'],
           p['e2t_W'], p['e2t_b'], pw1p, p['pos_b1'],
           p['pos_W2'], p['pos_b2']]
    tokpos = _encoder(gathered, cen16, wts)

    cls = jnp.broadcast_to(p['cls_token'] + p['cls_pos'], (_B, 1, _D_TR))
    x0 = jnp.concatenate([cls, tokpos.reshape(_B, _G, _D_TR)], axis=1)
    x0 = jnp.pad(x0, ((0, 0), (0, _TP - _T), (0, 0))).reshape(_B * _TP, _D_TR)
    emb = _transformer(x0, p['blocks'], p['norm_g'], p['norm_b'],
                       p['t2e_W'], p['t2e_b'])
    return emb[:_B]
